# baseline (speedup 1.0000x reference)
import functools

import jax
import jax.numpy as jnp
import numpy as np
from jax.experimental import pallas as pl
from jax.experimental.pallas import tpu as pltpu

_BLOCK_SIZE = 32
_BLOCK_COUNTS = 4


def _hstu_bsa_kernel(q_ref, k_ref, v_ref, gc_ref, gs_ref, maskbd_ref,
                     cbt_ref, poolbd_ref, causal_ref, gmat_ref, out_ref,
                     *, bs, ssel, H, D):
    f32 = jnp.float32
    bf = jnp.bfloat16
    hi = jax.lax.Precision.HIGHEST
    L = q_ref.shape[0]
    nblk = L // bs
    HD = H * D
    scale = D ** (-0.5)

    q_all = q_ref[...].reshape(L, HD)
    k_all = k_ref[...].reshape(L, HD)
    v_all = v_ref[...].reshape(L, HD)
    gc8 = gc_ref[...].reshape(L, H)
    gs8 = gs_ref[...].reshape(L, H)
    mask_bd = maskbd_ref[...]
    cbt = cbt_ref[...]
    pool_bd = poolbd_ref[...]
    causal_all = causal_ref[...]
    gmat = gmat_ref[...]

    tokp = jax.lax.broadcasted_iota(jnp.int32, (nblk, L), 1)
    blkp = jax.lax.broadcasted_iota(jnp.int32, (nblk, L), 0)
    pool = (tokp // bs == blkp).astype(f32)
    k_cmp = jnp.dot(pool, k_all, preferred_element_type=f32,
                    precision=hi) * (1.0 / bs)
    v_cmp = jnp.dot(pool, v_all, preferred_element_type=f32,
                    precision=hi) * (1.0 / bs)

    kcb = (jnp.concatenate([k_cmp] * H, axis=0) * mask_bd).astype(bf)
    vcb = (jnp.concatenate([v_cmp] * H, axis=0) * mask_bd).astype(bf)

    s_t = jax.lax.dot_general(
        kcb, q_all.astype(bf), (((1,), (1,)), ((), ())),
        preferred_element_type=f32) * scale
    s_sel = jnp.where(cbt > 0.5, s_t, -jnp.inf)

    s3 = s_sel.reshape(H, nblk, L)
    blk3 = jax.lax.broadcasted_iota(jnp.int32, (H, nblk, L), 1)
    rank = jnp.zeros((H, nblk, L), dtype=f32)
    for j in range(nblk):
        sj = jnp.broadcast_to(s3[:, j:j + 1, :], (H, nblk, L))
        beats = (sj > s3) | ((sj == s3) & (j < blk3))
        rank = rank + beats.astype(f32)
    sel_t = (rank < ssel).astype(f32).reshape(H * nblk, L)

    sel_q = jnp.transpose(sel_t).astype(bf)
    mask_all = jnp.dot(sel_q, pool_bd,
                       preferred_element_type=f32) * causal_all

    p_cmp_t = jnp.where(cbt > 0.5, s_t * jax.nn.sigmoid(s_t), 0.0)
    p_cmp = jnp.transpose(p_cmp_t).astype(bf)
    out_cmp = jnp.dot(p_cmp, vcb, preferred_element_type=f32)

    lane = jax.lax.broadcasted_iota(jnp.int32, (L, 2 * D), 1)
    m0 = (lane < D).astype(f32)
    m1 = 1.0 - m0
    outs = []
    for g in range(H // 2):
        sl = slice(g * 2 * D, (g + 1) * 2 * D)
        q2 = q_all[:, sl]
        k2 = k_all[:, sl]
        v2 = v_all[:, sl]
        k_st = jnp.concatenate([k2 * m0, k2 * m1], axis=0)
        v_st = jnp.concatenate([v2 * m0, v2 * m1], axis=0)
        s2 = jax.lax.dot_general(
            q2.astype(bf), k_st.astype(bf), (((1,), (1,)), ((), ())),
            preferred_element_type=f32) * scale
        p2 = s2 * jax.nn.sigmoid(s2) * mask_all[:, g * 2 * L:(g + 1) * 2 * L]
        outs.append(jnp.dot(p2.astype(bf), v_st.astype(bf),
                            preferred_element_type=f32))
    out_slc = jnp.concatenate(outs, axis=1)

    gc_b = jnp.dot(gc8, gmat, preferred_element_type=f32, precision=hi)
    gs_b = jnp.dot(gs8, gmat, preferred_element_type=f32, precision=hi)
    out_all = out_cmp * gc_b + out_slc * gs_b
    out_ref[...] = out_all.reshape(L, H, D)


def kernel(q, k, v, g_cmp, g_slc, x_offsets):
    T, H, D = q.shape
    Bn = x_offsets.shape[0] - 1
    L = T // Bn
    bs = _BLOCK_SIZE
    nblk = L // bs
    HD = H * D

    r_hb = np.arange(H * nblk)
    c_hd = np.arange(HD)
    mask_bd = (r_hb[:, None] // nblk == c_hd[None, :] // D).astype(np.float32)
    qpos = np.arange(L)
    cbt = (r_hb[:, None] % nblk <= qpos[None, :] // bs).astype(np.float32)
    c_ht = np.arange(H * L)
    pool_bd = jnp.asarray(
        (r_hb[:, None] // nblk == c_ht[None, :] // L)
        & (r_hb[:, None] % nblk == (c_ht[None, :] % L) // bs),
        dtype=jnp.bfloat16)
    causal_all = (c_ht[None, :] % L <= qpos[:, None]).astype(np.float32)
    gmat = (np.arange(H)[:, None] == c_hd[None, :] // D).astype(np.float32)

    def const_spec(shape):
        return pl.BlockSpec(shape, lambda b: (0,) * len(shape))

    body = functools.partial(_hstu_bsa_kernel, bs=bs,
                             ssel=min(_BLOCK_COUNTS, nblk), H=H, D=D)
    out = pl.pallas_call(
        body,
        grid=(Bn,),
        in_specs=[
            pl.BlockSpec((L, H, D), lambda b: (b, 0, 0)),
            pl.BlockSpec((L, H, D), lambda b: (b, 0, 0)),
            pl.BlockSpec((L, H, D), lambda b: (b, 0, 0)),
            pl.BlockSpec((L, H, 1), lambda b: (b, 0, 0)),
            pl.BlockSpec((L, H, 1), lambda b: (b, 0, 0)),
            const_spec((H * nblk, HD)),
            const_spec((H * nblk, L)),
            const_spec((H * nblk, H * L)),
            const_spec((L, H * L)),
            const_spec((H, HD)),
        ],
        out_specs=pl.BlockSpec((L, H, D), lambda b: (b, 0, 0)),
        out_shape=jax.ShapeDtypeStruct((T, H, D), jnp.float32),
        compiler_params=pltpu.CompilerParams(
            dimension_semantics=("parallel",)),
    )(q, k, v, g_cmp, g_slc, mask_bd, cbt, pool_bd, causal_all, gmat)

    return out


# speedup vs baseline: 1.2869x; 1.2869x over previous
import functools

import jax
import jax.numpy as jnp
import numpy as np
from jax.experimental import pallas as pl
from jax.experimental.pallas import tpu as pltpu

_BLOCK_SIZE = 32
_BLOCK_COUNTS = 4


def _hstu_bsa_kernel(q_ref, k_ref, v_ref, gc_ref, gs_ref, maskbd_ref,
                     cbt_ref, poolbd_ref, causal_ref, gmat_ref, out_ref,
                     *, bs, ssel, H, D):
    f32 = jnp.float32
    bf = jnp.bfloat16
    hi = jax.lax.Precision.HIGHEST
    L = q_ref.shape[0]
    nblk = L // bs
    HD = H * D
    scale = D ** (-0.5)

    q_all = q_ref[...]
    k_all = k_ref[...]
    v_all = v_ref[...]
    gc8 = gc_ref[...]
    gs8 = gs_ref[...]
    mask_bd = maskbd_ref[...]
    cbt = cbt_ref[...]
    pool_bd = poolbd_ref[...]
    causal_all = causal_ref[...]
    gmat = gmat_ref[...]

    tokp = jax.lax.broadcasted_iota(jnp.int32, (nblk, L), 1)
    blkp = jax.lax.broadcasted_iota(jnp.int32, (nblk, L), 0)
    pool = (tokp // bs == blkp).astype(f32)
    k_cmp = jnp.dot(pool, k_all, preferred_element_type=f32,
                    precision=hi) * (1.0 / bs)
    v_cmp = jnp.dot(pool, v_all, preferred_element_type=f32,
                    precision=hi) * (1.0 / bs)

    kcb = (jnp.concatenate([k_cmp] * H, axis=0) * mask_bd).astype(bf)
    vcb = (jnp.concatenate([v_cmp] * H, axis=0) * mask_bd).astype(bf)

    s_t = jax.lax.dot_general(
        kcb, q_all.astype(bf), (((1,), (1,)), ((), ())),
        preferred_element_type=f32) * scale
    s_sel = jnp.where(cbt > 0.5, s_t, -jnp.inf)

    s3 = s_sel.reshape(H, nblk, L)
    blk3 = jax.lax.broadcasted_iota(jnp.int32, (H, nblk, L), 1)
    rank = jnp.zeros((H, nblk, L), dtype=f32)
    for j in range(nblk):
        sj = jnp.broadcast_to(s3[:, j:j + 1, :], (H, nblk, L))
        beats = (sj > s3) | ((sj == s3) & (j < blk3))
        rank = rank + beats.astype(f32)
    sel_t = (rank < ssel).astype(f32).reshape(H * nblk, L)

    sel_q = jnp.transpose(sel_t).astype(bf)
    mask_all = jnp.dot(sel_q, pool_bd,
                       preferred_element_type=f32) * causal_all

    p_cmp_t = jnp.where(cbt > 0.5, s_t * jax.nn.sigmoid(s_t), 0.0)
    p_cmp = jnp.transpose(p_cmp_t).astype(bf)
    out_cmp = jnp.dot(p_cmp, vcb, preferred_element_type=f32)

    lane = jax.lax.broadcasted_iota(jnp.int32, (L, 2 * D), 1)
    m0 = (lane < D).astype(f32)
    m1 = 1.0 - m0
    outs = []
    for g in range(H // 2):
        sl = slice(g * 2 * D, (g + 1) * 2 * D)
        q2 = q_all[:, sl]
        k2 = k_all[:, sl]
        v2 = v_all[:, sl]
        k_st = jnp.concatenate([k2 * m0, k2 * m1], axis=0)
        v_st = jnp.concatenate([v2 * m0, v2 * m1], axis=0)
        s2 = jax.lax.dot_general(
            q2.astype(bf), k_st.astype(bf), (((1,), (1,)), ((), ())),
            preferred_element_type=f32) * scale
        p2 = s2 * jax.nn.sigmoid(s2) * mask_all[:, g * 2 * L:(g + 1) * 2 * L]
        outs.append(jnp.dot(p2.astype(bf), v_st.astype(bf),
                            preferred_element_type=f32))
    out_slc = jnp.concatenate(outs, axis=1)

    gc_b = jnp.dot(gc8, gmat, preferred_element_type=f32, precision=hi)
    gs_b = jnp.dot(gs8, gmat, preferred_element_type=f32, precision=hi)
    out_ref[...] = out_cmp * gc_b + out_slc * gs_b


def kernel(q, k, v, g_cmp, g_slc, x_offsets):
    T, H, D = q.shape
    Bn = x_offsets.shape[0] - 1
    L = T // Bn
    bs = _BLOCK_SIZE
    nblk = L // bs
    HD = H * D

    qf = q.reshape(T, HD)
    kf = k.reshape(T, HD)
    vf = v.reshape(T, HD)
    gcf = g_cmp.reshape(T, H)
    gsf = g_slc.reshape(T, H)

    r_hb = np.arange(H * nblk)
    c_hd = np.arange(HD)
    mask_bd = (r_hb[:, None] // nblk == c_hd[None, :] // D).astype(np.float32)
    qpos = np.arange(L)
    cbt = (r_hb[:, None] % nblk <= qpos[None, :] // bs).astype(np.float32)
    c_ht = np.arange(H * L)
    pool_bd = jnp.asarray(
        (r_hb[:, None] // nblk == c_ht[None, :] // L)
        & (r_hb[:, None] % nblk == (c_ht[None, :] % L) // bs),
        dtype=jnp.bfloat16)
    causal_all = (c_ht[None, :] % L <= qpos[:, None]).astype(np.float32)
    gmat = (np.arange(H)[:, None] == c_hd[None, :] // D).astype(np.float32)

    def const_spec(shape):
        return pl.BlockSpec(shape, lambda b: (0,) * len(shape))

    body = functools.partial(_hstu_bsa_kernel, bs=bs,
                             ssel=min(_BLOCK_COUNTS, nblk), H=H, D=D)
    out = pl.pallas_call(
        body,
        grid=(Bn,),
        in_specs=[
            pl.BlockSpec((L, HD), lambda b: (b, 0)),
            pl.BlockSpec((L, HD), lambda b: (b, 0)),
            pl.BlockSpec((L, HD), lambda b: (b, 0)),
            pl.BlockSpec((L, H), lambda b: (b, 0)),
            pl.BlockSpec((L, H), lambda b: (b, 0)),
            const_spec((H * nblk, HD)),
            const_spec((H * nblk, L)),
            const_spec((H * nblk, H * L)),
            const_spec((L, H * L)),
            const_spec((H, HD)),
        ],
        out_specs=pl.BlockSpec((L, HD), lambda b: (b, 0)),
        out_shape=jax.ShapeDtypeStruct((T, HD), jnp.float32),
        compiler_params=pltpu.CompilerParams(
            dimension_semantics=("parallel",)),
    )(qf, kf, vf, gcf, gsf, mask_bd, cbt, pool_bd, causal_all, gmat)

    return out.reshape(T, H, D)


# speedup vs baseline: 1.4350x; 1.1151x over previous
import functools

import jax
import jax.numpy as jnp
import numpy as np
from jax.experimental import pallas as pl
from jax.experimental.pallas import tpu as pltpu

_BLOCK_SIZE = 32
_BLOCK_COUNTS = 4


def _hstu_bsa_kernel(q_ref, k_ref, v_ref, gc_ref, gs_ref, maskbd_ref,
                     cbt_ref, poolbd_ref, causal_ref, gmat_ref, out_ref,
                     *, bs, ssel, H, D):
    f32 = jnp.float32
    bf = jnp.bfloat16
    hi = jax.lax.Precision.HIGHEST
    L = q_ref.shape[0]
    nblk = L // bs
    HD = H * D
    scale = D ** (-0.5)

    q_all = q_ref[...]
    k_all = k_ref[...]
    v_all = v_ref[...]
    gc8 = gc_ref[...]
    gs8 = gs_ref[...]
    mask_bd = maskbd_ref[...]
    cbt = cbt_ref[...]
    pool_bd = poolbd_ref[...]
    causal_all = causal_ref[...]
    gmat = gmat_ref[...]

    tokp = jax.lax.broadcasted_iota(jnp.int32, (nblk, L), 1)
    blkp = jax.lax.broadcasted_iota(jnp.int32, (nblk, L), 0)
    pool = (tokp // bs == blkp).astype(f32)
    k_cmp = jnp.dot(pool, k_all, preferred_element_type=f32,
                    precision=hi) * (1.0 / bs)
    v_cmp = jnp.dot(pool, v_all, preferred_element_type=f32,
                    precision=hi) * (1.0 / bs)

    kcb = (jnp.concatenate([k_cmp] * H, axis=0) * mask_bd).astype(bf)
    vcb = (jnp.concatenate([v_cmp] * H, axis=0) * mask_bd).astype(bf)

    s_t = jax.lax.dot_general(
        kcb, q_all.astype(bf), (((1,), (1,)), ((), ())),
        preferred_element_type=f32) * scale
    s_sel = jnp.where(cbt > 0.5, s_t, -jnp.inf)

    s3 = s_sel.reshape(H, nblk, L)
    blk3 = jax.lax.broadcasted_iota(jnp.int32, (H, nblk, L), 1)
    rank = jnp.zeros((H, nblk, L), dtype=f32)
    for j in range(nblk):
        sj = jnp.broadcast_to(s3[:, j:j + 1, :], (H, nblk, L))
        beats = (sj > s3) | ((sj == s3) & (j < blk3))
        rank = rank + beats.astype(f32)
    sel_t = (rank < ssel).astype(f32).reshape(H * nblk, L)

    sel_q = jnp.transpose(sel_t).astype(bf)
    mask_all = jnp.dot(sel_q, pool_bd,
                       preferred_element_type=f32) * causal_all

    p_cmp_t = jnp.where(cbt > 0.5, s_t * jax.nn.sigmoid(s_t), 0.0)
    p_cmp = jnp.transpose(p_cmp_t).astype(bf)
    out_cmp = jnp.dot(p_cmp, vcb, preferred_element_type=f32)

    lane = jax.lax.broadcasted_iota(jnp.int32, (L, 2 * D), 1)
    m0 = (lane < D).astype(f32)
    m1 = 1.0 - m0
    outs = []
    for g in range(H // 2):
        sl = slice(g * 2 * D, (g + 1) * 2 * D)
        q2 = q_all[:, sl]
        k2 = k_all[:, sl]
        v2 = v_all[:, sl]
        k_st = jnp.concatenate([k2 * m0, k2 * m1], axis=0)
        v_st = jnp.concatenate([v2 * m0, v2 * m1], axis=0)
        s2 = jax.lax.dot_general(
            q2.astype(bf), k_st.astype(bf), (((1,), (1,)), ((), ())),
            preferred_element_type=f32) * scale
        p2 = s2 * jax.nn.sigmoid(s2) * mask_all[:, g * 2 * L:(g + 1) * 2 * L]
        outs.append(jnp.dot(p2.astype(bf), v_st.astype(bf),
                            preferred_element_type=f32))
    out_slc = jnp.concatenate(outs, axis=1)

    gidx = jax.lax.broadcasted_iota(jnp.int32, (L, HD), 1) // D
    gc_b = jnp.take_along_axis(gc8, gidx, axis=1)
    gs_b = jnp.take_along_axis(gs8, gidx, axis=1)
    out_ref[...] = out_cmp * gc_b + out_slc * gs_b


def kernel(q, k, v, g_cmp, g_slc, x_offsets):
    T, H, D = q.shape
    Bn = x_offsets.shape[0] - 1
    L = T // Bn
    bs = _BLOCK_SIZE
    nblk = L // bs
    HD = H * D

    qf = q.reshape(T, HD)
    kf = k.reshape(T, HD)
    vf = v.reshape(T, HD)
    gcf = g_cmp.reshape(T, H)
    gsf = g_slc.reshape(T, H)

    r_hb = np.arange(H * nblk)
    c_hd = np.arange(HD)
    mask_bd = (r_hb[:, None] // nblk == c_hd[None, :] // D).astype(np.float32)
    qpos = np.arange(L)
    cbt = (r_hb[:, None] % nblk <= qpos[None, :] // bs).astype(np.float32)
    c_ht = np.arange(H * L)
    pool_bd = jnp.asarray(
        (r_hb[:, None] // nblk == c_ht[None, :] // L)
        & (r_hb[:, None] % nblk == (c_ht[None, :] % L) // bs),
        dtype=jnp.bfloat16)
    causal_all = (c_ht[None, :] % L <= qpos[:, None]).astype(np.float32)
    gmat = (np.arange(H)[:, None] == c_hd[None, :] // D).astype(np.float32)

    def const_spec(shape):
        return pl.BlockSpec(shape, lambda b: (0,) * len(shape))

    body = functools.partial(_hstu_bsa_kernel, bs=bs,
                             ssel=min(_BLOCK_COUNTS, nblk), H=H, D=D)
    out = pl.pallas_call(
        body,
        grid=(Bn,),
        in_specs=[
            pl.BlockSpec((L, HD), lambda b: (b, 0)),
            pl.BlockSpec((L, HD), lambda b: (b, 0)),
            pl.BlockSpec((L, HD), lambda b: (b, 0)),
            pl.BlockSpec((L, H), lambda b: (b, 0)),
            pl.BlockSpec((L, H), lambda b: (b, 0)),
            const_spec((H * nblk, HD)),
            const_spec((H * nblk, L)),
            const_spec((H * nblk, H * L)),
            const_spec((L, H * L)),
            const_spec((H, HD)),
        ],
        out_specs=pl.BlockSpec((L, HD), lambda b: (b, 0)),
        out_shape=jax.ShapeDtypeStruct((T, HD), jnp.float32),
        compiler_params=pltpu.CompilerParams(
            dimension_semantics=("parallel",)),
    )(qf, kf, vf, gcf, gsf, mask_bd, cbt, pool_bd, causal_all, gmat)

    return out.reshape(T, H, D)


# speedup vs baseline: 1.6177x; 1.1273x over previous
import functools

import jax
import jax.numpy as jnp
import numpy as np
from jax.experimental import pallas as pl
from jax.experimental.pallas import tpu as pltpu

_BLOCK_SIZE = 32
_BLOCK_COUNTS = 4


def _hstu_bsa_kernel(q_ref, k_ref, v_ref, gc_ref, gs_ref, maskbd_ref,
                     cbt_ref, poolbd_ref, causal_ref, gmat_ref, out_ref,
                     *, bs, ssel, H, D):
    f32 = jnp.float32
    bf = jnp.bfloat16
    hi = jax.lax.Precision.HIGHEST
    L = q_ref.shape[0]
    nblk = L // bs
    HD = H * D
    scale = D ** (-0.5)

    q_all = q_ref[...]
    k_all = k_ref[...]
    v_all = v_ref[...]
    gc8 = gc_ref[...]
    gs8 = gs_ref[...]
    mask_bd = maskbd_ref[...]
    cbt = cbt_ref[...]
    pool_bd = poolbd_ref[...]
    causal2 = causal_ref[...]
    gmat = gmat_ref[...]

    k_cmp = jnp.sum(k_all.reshape(nblk, bs, HD), axis=1) * (1.0 / bs)
    v_cmp = jnp.sum(v_all.reshape(nblk, bs, HD), axis=1) * (1.0 / bs)

    kcb = (jnp.concatenate([k_cmp] * H, axis=0) * mask_bd).astype(bf)
    vcb = (jnp.concatenate([v_cmp] * H, axis=0) * mask_bd).astype(bf)

    s_t = jax.lax.dot_general(
        kcb, q_all, (((1,), (1,)), ((), ())),
        preferred_element_type=f32) * scale
    s_sel = jnp.where(cbt > 0.5, s_t, -jnp.inf)

    s3 = s_sel.reshape(H, nblk, L)
    blk3 = jax.lax.broadcasted_iota(jnp.int32, (H, nblk, L), 1)
    rank = jnp.zeros((H, nblk, L), dtype=f32)
    for j in range(nblk):
        sj = jnp.broadcast_to(s3[:, j:j + 1, :], (H, nblk, L))
        beats = (sj > s3) | ((sj == s3) & (j < blk3))
        rank = rank + beats.astype(f32)
    sel_t = (rank < ssel).astype(f32).reshape(H * nblk, L)

    sel_q = jnp.transpose(sel_t).astype(bf)
    mask_all = jnp.dot(sel_q, pool_bd, preferred_element_type=f32)

    p_cmp_t = jnp.where(cbt > 0.5, s_t * jax.nn.sigmoid(s_t), 0.0)
    p_cmp = jnp.transpose(p_cmp_t).astype(bf)
    out_cmp = jnp.dot(p_cmp, vcb, preferred_element_type=f32)

    lane = jax.lax.broadcasted_iota(jnp.int32, (L, 2 * D), 1)
    m0 = (lane < D).astype(f32)
    m1 = 1.0 - m0
    outs = []
    for g in range(H // 2):
        sl = slice(g * 2 * D, (g + 1) * 2 * D)
        q2 = q_all[:, sl]
        k2 = k_all[:, sl]
        v2 = v_all[:, sl]
        k_st = jnp.concatenate([k2 * m0, k2 * m1], axis=0)
        v_st = jnp.concatenate([v2 * m0, v2 * m1], axis=0)
        s2 = jax.lax.dot_general(
            q2, k_st.astype(bf), (((1,), (1,)), ((), ())),
            preferred_element_type=f32) * scale
        p2 = (s2 * jax.nn.sigmoid(s2) * causal2
              * mask_all[:, g * 2 * L:(g + 1) * 2 * L])
        outs.append(jnp.dot(p2.astype(bf), v_st.astype(bf),
                            preferred_element_type=f32))
    out_slc = jnp.concatenate(outs, axis=1)

    gidx = jax.lax.broadcasted_iota(jnp.int32, (L, HD), 1) // D
    gc_b = jnp.take_along_axis(gc8, gidx, axis=1)
    gs_b = jnp.take_along_axis(gs8, gidx, axis=1)
    out_ref[...] = out_cmp * gc_b + out_slc * gs_b


def kernel(q, k, v, g_cmp, g_slc, x_offsets):
    T, H, D = q.shape
    Bn = x_offsets.shape[0] - 1
    L = T // Bn
    bs = _BLOCK_SIZE
    nblk = L // bs
    HD = H * D

    qf = q.reshape(T, HD).astype(jnp.bfloat16)
    kf = k.reshape(T, HD)
    vf = v.reshape(T, HD)
    gcf = g_cmp.reshape(T, H)
    gsf = g_slc.reshape(T, H)

    r_hb = np.arange(H * nblk)
    c_hd = np.arange(HD)
    mask_bd = (r_hb[:, None] // nblk == c_hd[None, :] // D).astype(np.float32)
    qpos = np.arange(L)
    cbt = (r_hb[:, None] % nblk <= qpos[None, :] // bs).astype(np.float32)
    c_ht = np.arange(H * L)
    pool_bd = jnp.asarray(
        (r_hb[:, None] // nblk == c_ht[None, :] // L)
        & (r_hb[:, None] % nblk == (c_ht[None, :] % L) // bs),
        dtype=jnp.bfloat16)
    c_2t = np.arange(2 * L)
    causal2 = (c_2t[None, :] % L <= qpos[:, None]).astype(np.float32)
    gmat = (np.arange(H)[:, None] == c_hd[None, :] // D).astype(np.float32)

    def const_spec(shape):
        return pl.BlockSpec(shape, lambda b: (0,) * len(shape))

    body = functools.partial(_hstu_bsa_kernel, bs=bs,
                             ssel=min(_BLOCK_COUNTS, nblk), H=H, D=D)
    out = pl.pallas_call(
        body,
        grid=(Bn,),
        in_specs=[
            pl.BlockSpec((L, HD), lambda b: (b, 0)),
            pl.BlockSpec((L, HD), lambda b: (b, 0)),
            pl.BlockSpec((L, HD), lambda b: (b, 0)),
            pl.BlockSpec((L, H), lambda b: (b, 0)),
            pl.BlockSpec((L, H), lambda b: (b, 0)),
            const_spec((H * nblk, HD)),
            const_spec((H * nblk, L)),
            const_spec((H * nblk, H * L)),
            const_spec((L, 2 * L)),
            const_spec((H, HD)),
        ],
        out_specs=pl.BlockSpec((L, HD), lambda b: (b, 0)),
        out_shape=jax.ShapeDtypeStruct((T, HD), jnp.float32),
        compiler_params=pltpu.CompilerParams(
            dimension_semantics=("parallel",)),
    )(qf, kf, vf, gcf, gsf, mask_bd, cbt, pool_bd, causal2, gmat)

    return out.reshape(T, H, D)
